# Initial kernel scaffold; baseline (speedup 1.0000x reference)
import jax
import jax.numpy as jnp
from jax.experimental import pallas as pl


def kernel(box_cls, box_regression, anchors):
    raise NotImplementedError("write your pallas kernel here")


# speedup vs baseline: 2.7425x; 2.7425x over previous
import functools

import jax
import jax.numpy as jnp
from jax import lax
from jax.experimental import pallas as pl
from jax.experimental.pallas import tpu as pltpu
from jax.experimental.pallas import tpu_sc as plsc

INFER_TH = 0.05
TOPK = 5000
NHWC = 160000
ROWS = 1280
LANES = 128
RB = ROWS // LANES
KPAD = 5120
BLK = 512
NB = KPAD // BLK
HI_P = lax.Precision.HIGHEST


def _select_body(cls_ref, sm_ref, mf_ref, rb_ref):
    x = cls_ref[0, 0, :].reshape(ROWS, LANES)
    row_i = lax.broadcasted_iota(jnp.int32, (ROWS, LANES), 0)
    lane_i = lax.broadcasted_iota(jnp.int32, (ROWS, LANES), 1)
    flat_i = row_i * LANES + lane_i
    in_range = flat_i < NHWC

    s = jax.nn.sigmoid(x)
    s = jnp.where((s > INFER_TH) & in_range, s, 0.0)

    def cnt_gt(v):
        return jnp.sum((s > v).astype(jnp.int32))

    def bs_body(_, carry):
        lo, hi = carry
        mid = (lo + hi) // 2
        midf = lax.bitcast_convert_type(mid, jnp.float32)
        pred = cnt_gt(midf) < TOPK
        return jnp.where(pred, lo, mid + 1), jnp.where(pred, mid, hi)

    _, vstar_bits = lax.fori_loop(
        0, 31, bs_body, (jnp.int32(0), jnp.int32(0x3F800000)))
    vstar = lax.bitcast_convert_type(vstar_bits, jnp.float32)

    n_gt = cnt_gt(vstar)
    n_tie_needed = TOPK - n_gt
    is_tie = (s == vstar) & in_range

    def cnt_tie_below(b):
        return jnp.sum((is_tie & (flat_i < b)).astype(jnp.int32))

    def bs2_body(_, carry):
        lo, hi = carry
        mid = (lo + hi) // 2
        pred = cnt_tie_below(mid) >= n_tie_needed
        return jnp.where(pred, lo, mid + 1), jnp.where(pred, mid, hi)

    _, bnd = lax.fori_loop(0, 18, bs2_body, (jnp.int32(0), jnp.int32(NHWC)))

    m = (s > vstar) | (is_tie & (flat_i < bnd))
    mf = m.astype(jnp.float32)

    sm_ref[0, 0, :] = jnp.where(m, s, 0.0).reshape(ROWS * LANES)
    mf_ref[0, 0, :] = mf.reshape(ROWS * LANES)

    r_cnt = jnp.sum(mf, axis=1).reshape(RB, LANES)
    li = lax.broadcasted_iota(jnp.int32, (LANES, LANES), 0)
    lj = lax.broadcasted_iota(jnp.int32, (LANES, LANES), 1)
    tl_excl = (li < lj).astype(jnp.float32)
    lane_exc = jnp.dot(r_cnt, tl_excl, precision=HI_P,
                       preferred_element_type=jnp.float32)
    blk_tot = jnp.sum(r_cnt, axis=1, keepdims=True)
    bi = lax.broadcasted_iota(jnp.int32, (RB, RB), 0)
    bj = lax.broadcasted_iota(jnp.int32, (RB, RB), 1)
    tb_excl = (bj < bi).astype(jnp.float32)
    blk_exc = jnp.dot(tb_excl, blk_tot, precision=HI_P,
                      preferred_element_type=jnp.float32)
    rb_ref[0, 0, :] = (lane_exc + blk_exc).reshape(ROWS)


def _run_select(cls_pad):
    N = cls_pad.shape[0]
    return pl.pallas_call(
        _select_body,
        grid=(N,),
        in_specs=[pl.BlockSpec((1, 1, ROWS * LANES), lambda n: (n, 0, 0))],
        out_specs=[
            pl.BlockSpec((1, 1, ROWS * LANES), lambda n: (n, 0, 0)),
            pl.BlockSpec((1, 1, ROWS * LANES), lambda n: (n, 0, 0)),
            pl.BlockSpec((1, 1, ROWS), lambda n: (n, 0, 0)),
        ],
        out_shape=[
            jax.ShapeDtypeStruct((N, 1, ROWS * LANES), jnp.float32),
            jax.ShapeDtypeStruct((N, 1, ROWS * LANES), jnp.float32),
            jax.ShapeDtypeStruct((N, 1, ROWS), jnp.float32),
        ],
    )(cls_pad)


def _compact_body(sm_ref, mf_ref, rb_ref, ckey_ref, cidx_ref):
    b = pl.program_id(1)
    s = sm_ref[0, 0, :].reshape(ROWS, LANES)
    mf = mf_ref[0, 0, :].reshape(ROWS, LANES)
    row_base = rb_ref[0, 0, :].reshape(1, ROWS)

    q = (lax.broadcasted_iota(jnp.int32, (BLK, 1), 0)
         + b * BLK).astype(jnp.float32)
    cmp = (row_base <= q).astype(jnp.float32)
    ros = jnp.sum(cmp, axis=1, keepdims=True).astype(jnp.int32) - 1
    onehot = (lax.broadcasted_iota(jnp.int32, (BLK, ROWS), 1)
              == ros).astype(jnp.float32)
    row_base_g = jnp.dot(onehot, row_base.reshape(ROWS, 1), precision=HI_P,
                         preferred_element_type=jnp.float32)
    s_rows = jnp.dot(onehot, s, precision=HI_P,
                     preferred_element_type=jnp.float32)
    m_rows = jnp.dot(onehot, mf, precision=HI_P,
                     preferred_element_type=jnp.float32)

    li = lax.broadcasted_iota(jnp.int32, (LANES, LANES), 0)
    lj = lax.broadcasted_iota(jnp.int32, (LANES, LANES), 1)
    tl_excl = (li < lj).astype(jnp.float32)
    lane_pref = jnp.dot(m_rows, tl_excl, precision=HI_P,
                        preferred_element_type=jnp.float32)

    t_q = q - row_base_g
    ind = (m_rows > 0.5) & (lane_pref == t_q)
    ckey_ref[0, :, :] = jnp.sum(jnp.where(ind, s_rows, 0.0),
                                axis=1, keepdims=True)
    lane_idx = lax.broadcasted_iota(jnp.int32, (BLK, LANES), 1)
    cidx_ref[0, :, :] = jnp.sum(jnp.where(ind, ros * LANES + lane_idx, 0),
                                axis=1, keepdims=True)


def _run_compact(sm, mf, rbase):
    N = sm.shape[0]
    return pl.pallas_call(
        _compact_body,
        grid=(N, NB),
        in_specs=[
            pl.BlockSpec((1, 1, ROWS * LANES), lambda n, b: (n, 0, 0)),
            pl.BlockSpec((1, 1, ROWS * LANES), lambda n, b: (n, 0, 0)),
            pl.BlockSpec((1, 1, ROWS), lambda n, b: (n, 0, 0)),
        ],
        out_specs=[
            pl.BlockSpec((1, BLK, 1), lambda n, b: (n, b, 0)),
            pl.BlockSpec((1, BLK, 1), lambda n, b: (n, b, 0)),
        ],
        out_shape=[
            jax.ShapeDtypeStruct((N, KPAD, 1), jnp.float32),
            jax.ShapeDtypeStruct((N, KPAD, 1), jnp.int32),
        ],
    )(sm, mf, rbase)


def _rank_body(ckey_ref, cidx_ref, ckb_ref, cib_ref, rank_ref):
    b = pl.program_id(1)

    @pl.when(b == 0)
    def _():
        rank_ref[0, :, :] = jnp.zeros((KPAD, 1), jnp.float32)

    ckey = ckey_ref[0, :, :]
    cidx = cidx_ref[0, :, :]
    kb = ckb_ref[0, 0, :].reshape(1, BLK)
    ib = cib_ref[0, 0, :].reshape(1, BLK)
    valid = (b * BLK
             + lax.broadcasted_iota(jnp.int32, (1, BLK), 1)) < TOPK
    beats = (kb > ckey) | ((kb == ckey) & (ib < cidx))
    contrib = jnp.sum(jnp.where(beats & valid, 1.0, 0.0),
                      axis=1, keepdims=True)
    rank_ref[0, :, :] += contrib


def _run_rank(ckey, cidx):
    N = ckey.shape[0]
    ckey_row = ckey.reshape(N, 1, KPAD)
    cidx_row = cidx.reshape(N, 1, KPAD)
    return pl.pallas_call(
        _rank_body,
        grid=(N, NB),
        in_specs=[
            pl.BlockSpec((1, KPAD, 1), lambda n, b: (n, 0, 0)),
            pl.BlockSpec((1, KPAD, 1), lambda n, b: (n, 0, 0)),
            pl.BlockSpec((1, 1, BLK), lambda n, b: (n, 0, b)),
            pl.BlockSpec((1, 1, BLK), lambda n, b: (n, 0, b)),
        ],
        out_specs=pl.BlockSpec((1, KPAD, 1), lambda n, b: (n, 0, 0)),
        out_shape=jax.ShapeDtypeStruct((N, KPAD, 1), jnp.float32),
    )(ckey, cidx, ckey_row, cidx_row)


def _apply_body(rkb_ref, ckb_ref, cib_ref, vals_ref, idx_ref):
    b = pl.program_id(1)

    @pl.when(b == 0)
    def _():
        vals_ref[0, :, :] = jnp.zeros((KPAD, 1), jnp.float32)
        idx_ref[0, :, :] = jnp.zeros((KPAD, 1), jnp.int32)

    rb = rkb_ref[0, 0, :].reshape(1, BLK)
    kb = ckb_ref[0, 0, :].reshape(1, BLK)
    ib = cib_ref[0, 0, :].reshape(1, BLK)
    valid = (b * BLK
             + lax.broadcasted_iota(jnp.int32, (1, BLK), 1)) < TOPK
    q = lax.broadcasted_iota(jnp.int32, (KPAD, 1), 0).astype(jnp.float32)
    hit = (rb == q) & valid
    vals_ref[0, :, :] += jnp.sum(jnp.where(hit, kb, 0.0),
                                 axis=1, keepdims=True)
    idx_ref[0, :, :] += jnp.sum(jnp.where(hit, ib, 0),
                                axis=1, keepdims=True)


def _run_apply(rank, ckey, cidx):
    N = ckey.shape[0]
    return pl.pallas_call(
        _apply_body,
        grid=(N, NB),
        in_specs=[
            pl.BlockSpec((1, 1, BLK), lambda n, b: (n, 0, b)),
            pl.BlockSpec((1, 1, BLK), lambda n, b: (n, 0, b)),
            pl.BlockSpec((1, 1, BLK), lambda n, b: (n, 0, b)),
        ],
        out_specs=[
            pl.BlockSpec((1, KPAD, 1), lambda n, b: (n, 0, 0)),
            pl.BlockSpec((1, KPAD, 1), lambda n, b: (n, 0, 0)),
        ],
        out_shape=[
            jax.ShapeDtypeStruct((N, KPAD, 1), jnp.float32),
            jax.ShapeDtypeStruct((N, KPAD, 1), jnp.int32),
        ],
    )(rank.reshape(N, 1, KPAD), ckey.reshape(N, 1, KPAD),
      cidx.reshape(N, 1, KPAD))


def _make_sc_gather(n_img):
    info = plsc.get_sparse_core_info()
    nw = info.num_cores * info.num_subcores
    btot = n_img * KPAD
    bpw = btot // nw
    nc = info.num_cores
    mesh = plsc.VectorSubcoreMesh(core_axis_name="c", subcore_axis_name="s")

    @functools.partial(
        pl.kernel,
        mesh=mesh,
        out_type=[
            jax.ShapeDtypeStruct((btot, 16), jnp.float32),
            jax.ShapeDtypeStruct((btot, 8), jnp.float32),
        ],
        scratch_types=[
            pltpu.VMEM((bpw,), jnp.int32),
            pltpu.VMEM((bpw, 16), jnp.float32),
            pltpu.VMEM((bpw,), jnp.int32),
            pltpu.VMEM((bpw, 8), jnp.float32),
            pltpu.SemaphoreType.DMA,
            pltpu.SemaphoreType.DMA,
        ],
        compiler_params=pltpu.CompilerParams(use_tc_tiling_on_sc=False),
    )
    def gather_k(reg_hbm, anc_hbm, ridx_hbm, lidx_hbm, reg_out, anc_out,
                 ridx_v, rrows_v, lidx_v, arows_v, sem1, sem2):
        wid = lax.axis_index("s") * nc + lax.axis_index("c")
        base = wid * bpw
        pltpu.sync_copy(ridx_hbm.at[pl.ds(base, bpw)], ridx_v)
        pltpu.sync_copy(lidx_hbm.at[pl.ds(base, bpw)], lidx_v)
        cp1 = pltpu.async_copy(reg_hbm.at[ridx_v], rrows_v, sem1)
        cp2 = pltpu.async_copy(anc_hbm.at[lidx_v], arows_v, sem2)
        cp1.wait()
        cp2.wait()
        pltpu.sync_copy(rrows_v, reg_out.at[pl.ds(base, bpw)])
        pltpu.sync_copy(arows_v, anc_out.at[pl.ds(base, bpw)])

    return gather_k


def _decode_body(reg_ref, anc_ref, vals_ref, idx_ref, det_ref, lab_ref,
                 sc_ref):
    r = reg_ref[0]
    a = anc_ref[0]
    x1 = a[:, 0:1]
    y1 = a[:, 1:2]
    w = a[:, 2:3] - x1
    h = a[:, 3:4] - y1
    cx = x1 + 0.5 * w
    cy = y1 + 0.5 * h
    kp_x = cx + r[:, :8] * w
    kp_y = cy + r[:, 8:] * h
    det_ref[0] = jnp.concatenate([kp_x, kp_y], axis=-1)
    lab_ref[0] = idx_ref[0] % 16 + 1
    sc_ref[0] = jnp.sqrt(vals_ref[0] + 1e-12)


def _run_decode(reg_rows, anc_rows, vals, idx):
    N = reg_rows.shape[0]
    return pl.pallas_call(
        _decode_body,
        grid=(N,),
        in_specs=[
            pl.BlockSpec((1, KPAD, 16), lambda n: (n, 0, 0)),
            pl.BlockSpec((1, KPAD, 8), lambda n: (n, 0, 0)),
            pl.BlockSpec((1, KPAD, 1), lambda n: (n, 0, 0)),
            pl.BlockSpec((1, KPAD, 1), lambda n: (n, 0, 0)),
        ],
        out_specs=[
            pl.BlockSpec((1, KPAD, 16), lambda n: (n, 0, 0)),
            pl.BlockSpec((1, KPAD, 1), lambda n: (n, 0, 0)),
            pl.BlockSpec((1, KPAD, 1), lambda n: (n, 0, 0)),
        ],
        out_shape=[
            jax.ShapeDtypeStruct((N, KPAD, 16), jnp.float32),
            jax.ShapeDtypeStruct((N, KPAD, 1), jnp.int32),
            jax.ShapeDtypeStruct((N, KPAD, 1), jnp.float32),
        ],
    )(reg_rows, anc_rows, vals, idx)


def kernel(box_cls, box_regression, anchors):
    N, C, H, W = box_cls.shape
    HW = H * W
    cls_flat = jnp.transpose(box_cls, (0, 2, 3, 1)).reshape(N, 1, HW * C)
    cls_pad = jnp.pad(cls_flat, ((0, 0), (0, 0), (0, ROWS * LANES - HW * C)))

    sm, mf, rbase = _run_select(cls_pad)
    ckey, cidx = _run_compact(sm, mf, rbase)
    rank = _run_rank(ckey, cidx)
    vals, idx = _run_apply(rank, ckey, cidx)

    sidx = idx.reshape(N, KPAD)
    img_off = (jnp.arange(N, dtype=jnp.int32) * (HW * C))[:, None]
    ridx = (sidx + img_off).reshape(N * KPAD)
    loc_off = (jnp.arange(N, dtype=jnp.int32) * HW)[:, None]
    lidx = (sidx // C + loc_off).reshape(N * KPAD)

    reg_table = jnp.transpose(box_regression, (0, 2, 3, 1)).reshape(
        N * HW * C, 16)
    anc_table = jnp.pad(anchors.reshape(N * HW, 4), ((0, 0), (0, 4)))

    reg_rows, anc_rows = _make_sc_gather(N)(reg_table, anc_table, ridx, lidx)
    det_p, lab_p, sc_p = _run_decode(
        reg_rows.reshape(N, KPAD, 16), anc_rows.reshape(N, KPAD, 8),
        vals, idx)
    det = det_p[:, :TOPK, :]
    labels = lab_p.reshape(N, KPAD)[:, :TOPK]
    scores = sc_p.reshape(N, KPAD)[:, :TOPK]
    return det, labels, scores
